# Initial kernel scaffold; baseline (speedup 1.0000x reference)
import jax
import jax.numpy as jnp
from jax.experimental import pallas as pl


def kernel(claim, targets, embeddings):
    raise NotImplementedError("write your pallas kernel here")


# speedup vs baseline: 2.8186x; 2.8186x over previous
import functools

import jax
import jax.numpy as jnp
from jax import lax
from jax.experimental import pallas as pl
from jax.experimental.pallas import tpu as pltpu
from jax.experimental.pallas import tpu_sc as plsc

_NC = 2
_NS = 16
_NW = _NC * _NS


def _sc_gather(src, ids, chunk):
    M = ids.shape[0]
    D = src.shape[1]
    per_w = M // _NW
    assert per_w * _NW == M and per_w % chunk == 0
    n_chunks = per_w // chunk
    mesh = plsc.VectorSubcoreMesh(
        core_axis_name="c", subcore_axis_name="s", num_cores=_NC,
        num_subcores=_NS)

    @functools.partial(
        pl.kernel,
        mesh=mesh,
        compiler_params=pltpu.CompilerParams(use_tc_tiling_on_sc=False),
        out_type=jax.ShapeDtypeStruct((M, D), jnp.float32),
        scratch_types=[
            pltpu.VMEM((chunk,), jnp.int32),
            pltpu.VMEM((chunk, D), jnp.float32),
            pltpu.SemaphoreType.DMA,
        ],
    )
    def gather_kernel(ids_hbm, src_hbm, out_hbm, idx_v, rows_v, sem):
        wid = lax.axis_index("s") * _NC + lax.axis_index("c")
        base = wid * per_w

        def body(i, carry):
            off = base + i * chunk
            pltpu.sync_copy(ids_hbm.at[pl.ds(off, chunk)], idx_v)
            pltpu.async_copy(src_hbm.at[idx_v], rows_v, sem).wait()
            pltpu.sync_copy(rows_v, out_hbm.at[pl.ds(off, chunk)])
            return carry

        lax.fori_loop(0, n_chunks, body, 0)

    return gather_kernel(ids, src)


def _score_topk_kernel(t_ref, c_ref, o_ref, ts_ref):
    T, LT, LC = 512, 32, 16
    N = 64
    TC = 128
    c = c_ref[0]
    for ck in range(T // TC):
        t = t_ref[0, pl.ds(ck * TC * LT, TC * LT), :]
        u = lax.dot_general(t, c, (((1,), (1,)), ((), ())),
                            preferred_element_type=jnp.float32)
        u3 = u.reshape(TC, LT, LC)
        m = jnp.max(u3, axis=1, keepdims=True)
        e = jnp.exp(u3 - m)
        den = jnp.sum(e, axis=1, keepdims=True)
        sm = e / den
        sc = jnp.max(sm, axis=2)
        ts_ref[0, pl.ds(ck * TC, TC)] = jnp.sum(sc, axis=1)
    ts = ts_ref[0, :]

    tsr = ts[:, None]
    tsc = ts[None, :]
    row_i = lax.broadcasted_iota(jnp.int32, (T, T), 0)
    col_i = lax.broadcasted_iota(jnp.int32, (T, T), 1)
    gt = (tsc > tsr) | ((tsc == tsr) & (col_i < row_i))
    rank = jnp.sum(gt.astype(jnp.int32), axis=1)

    kio = lax.broadcasted_iota(jnp.int32, (N, T), 0)
    jio = lax.broadcasted_iota(jnp.int32, (N, T), 1)
    sel = rank[None, :] == kio
    perm = jnp.sum(jnp.where(sel, jio, 0), axis=1)

    b = pl.program_id(0)
    lio = lax.broadcasted_iota(jnp.int32, (N, LT), 1)
    o_ref[0] = b * (T * LT) + perm[:, None] * LT + lio


def _finalize_kernel(t_ref, c_ref, o_ref):
    N, LT, LC = 64, 32, 16
    t = t_ref[0]
    c = c_ref[0]
    u = lax.dot_general(t, c, (((1,), (1,)), ((), ())),
                        preferred_element_type=jnp.float32)
    u3 = u.reshape(N, LT, LC)
    nrm = jnp.sqrt(jnp.sum(u3 * u3, axis=1, keepdims=True))
    o_ref[0] = u3 / nrm


def kernel(claim, targets, embeddings):
    B, LC = claim.shape
    _, T, LT = targets.shape
    D = embeddings.shape[1]
    N = 64

    t_ids = targets.reshape(-1).astype(jnp.int32)
    c_ids = claim.reshape(-1).astype(jnp.int32)

    t_g = _sc_gather(embeddings, t_ids, chunk=1024)
    c_g = _sc_gather(embeddings, c_ids, chunk=B * LC // _NW)

    t_g3 = t_g.reshape(B, T * LT, D)
    c_g3 = c_g.reshape(B, LC, D)

    idxf = pl.pallas_call(
        _score_topk_kernel,
        grid=(B,),
        in_specs=[
            pl.BlockSpec((1, T * LT, D), lambda b: (b, 0, 0)),
            pl.BlockSpec((1, LC, D), lambda b: (b, 0, 0)),
        ],
        out_specs=pl.BlockSpec((1, N, LT), lambda b: (b, 0, 0)),
        out_shape=jax.ShapeDtypeStruct((B, N, LT), jnp.int32),
        scratch_shapes=[pltpu.VMEM((1, T), jnp.float32)],
    )(t_g3, c_g3)

    sel_ids = idxf.reshape(-1)
    t_sel = _sc_gather(t_g, sel_ids, chunk=B * N * LT // _NW)

    fin = pl.pallas_call(
        _finalize_kernel,
        grid=(B,),
        in_specs=[
            pl.BlockSpec((1, N * LT, D), lambda b: (b, 0, 0)),
            pl.BlockSpec((1, LC, D), lambda b: (b, 0, 0)),
        ],
        out_specs=pl.BlockSpec((1, N, LT, LC), lambda b: (b, 0, 0, 0)),
        out_shape=jax.ShapeDtypeStruct((B, N, LT, LC), jnp.float32),
    )(t_sel.reshape(B, N * LT, D), c_g3)

    return fin.transpose(0, 1, 3, 2)


# speedup vs baseline: 3.2481x; 1.1524x over previous
import functools

import jax
import jax.numpy as jnp
from jax import lax
from jax.experimental import pallas as pl
from jax.experimental.pallas import tpu as pltpu
from jax.experimental.pallas import tpu_sc as plsc

_NC = 2
_NS = 16
_NW = _NC * _NS

_T = 512
_LT = 32
_LC = 16
_N = 64
_TJ = 128
_NCK = _T // _TJ
_CHJ = 64


def _sc_gather_all(emb, targets, claim):
    B, T, LT = targets.shape
    _, LC = claim.shape
    D = emb.shape[1]
    M = B * T * LT
    jw = T // 2
    n_steps = jw // _CHJ
    cw = B * LC // _NW
    mesh = plsc.VectorSubcoreMesh(
        core_axis_name="c", subcore_axis_name="s", num_cores=_NC,
        num_subcores=_NS)

    @functools.partial(
        pl.kernel,
        mesh=mesh,
        compiler_params=pltpu.CompilerParams(use_tc_tiling_on_sc=False),
        out_type=(jax.ShapeDtypeStruct((M, D), jnp.float32),
                  jax.ShapeDtypeStruct((B * LC, D), jnp.float32)),
        scratch_types=[
            pltpu.VMEM((_CHJ, LT), jnp.int32),
            pltpu.VMEM((_CHJ * LT,), jnp.int32),
            pltpu.VMEM((_CHJ * LT,), jnp.int32),
            pltpu.VMEM((_CHJ * LT, D), jnp.float32),
            pltpu.VMEM((cw,), jnp.int32),
            pltpu.VMEM((cw, D), jnp.float32),
            pltpu.SemaphoreType.DMA,
        ],
    )
    def gather_kernel(tgt_hbm, clm_hbm, src_hbm, out_hbm, outc_hbm,
                      idx2_v, idx1_v, dst_v, rows_v, cidx_v, crows_v, sem):
        wid = lax.axis_index("s") * _NC + lax.axis_index("c")
        b0 = wid // 2
        half = wid % 2

        cbase = wid * cw
        pltpu.sync_copy(clm_hbm.at[b0, pl.ds(half * cw, cw)], cidx_v)
        pltpu.async_copy(src_hbm.at[cidx_v], crows_v, sem).wait()
        pltpu.sync_copy(crows_v, outc_hbm.at[pl.ds(cbase, cw)])

        j0 = half * jw
        lane = lax.iota(jnp.int32, 16)

        def body(i, carry):
            pltpu.sync_copy(tgt_hbm.at[b0, pl.ds(j0 + i * _CHJ, _CHJ), :],
                            idx2_v)

            def unpack(r, c2):
                j = j0 + i * _CHJ + r
                s = (b0 * (T * LT) + (j // _TJ) * (_LT * _TJ) + (j % _TJ))
                for c in range(LT // 16):
                    co = c * 16
                    idx1_v[pl.ds(r * LT + co, 16)] = idx2_v[r, pl.ds(co, 16)]
                    dst_v[pl.ds(r * LT + co, 16)] = s + (lane + co) * _TJ
                return c2

            lax.fori_loop(0, _CHJ, unpack, 0)
            pltpu.async_copy(src_hbm.at[idx1_v], rows_v, sem).wait()
            pltpu.async_copy(rows_v, out_hbm.at[dst_v], sem).wait()
            return carry

        lax.fori_loop(0, n_steps, body, 0)

    return gather_kernel(targets, claim, emb)


def _sc_gather(src, ids, chunk):
    M = ids.shape[0]
    D = src.shape[1]
    per_w = M // _NW
    n_chunks = per_w // chunk
    mesh = plsc.VectorSubcoreMesh(
        core_axis_name="c", subcore_axis_name="s", num_cores=_NC,
        num_subcores=_NS)

    @functools.partial(
        pl.kernel,
        mesh=mesh,
        compiler_params=pltpu.CompilerParams(use_tc_tiling_on_sc=False),
        out_type=jax.ShapeDtypeStruct((M, D), jnp.float32),
        scratch_types=[
            pltpu.VMEM((chunk,), jnp.int32),
            pltpu.VMEM((chunk, D), jnp.float32),
            pltpu.SemaphoreType.DMA,
        ],
    )
    def gather_kernel(ids_hbm, src_hbm, out_hbm, idx_v, rows_v, sem):
        wid = lax.axis_index("s") * _NC + lax.axis_index("c")
        base = wid * per_w

        def body(i, carry):
            off = base + i * chunk
            pltpu.sync_copy(ids_hbm.at[pl.ds(off, chunk)], idx_v)
            pltpu.async_copy(src_hbm.at[idx_v], rows_v, sem).wait()
            pltpu.sync_copy(rows_v, out_hbm.at[pl.ds(off, chunk)])
            return carry

        lax.fori_loop(0, n_chunks, body, 0)

    return gather_kernel(ids, src)


def _score_topk_kernel(t_ref, c_ref, o_ref, ts_ref):
    c = c_ref[...]
    for ck in range(_NCK):
        t = t_ref[pl.ds(ck * _LT * _TJ, _LT * _TJ), :]
        u = lax.dot_general(c, t, (((1,), (1,)), ((), ())),
                            preferred_element_type=jnp.float32)
        u3 = u.reshape(_LC, _LT, _TJ)
        m = jnp.max(u3, axis=1, keepdims=True)
        e = jnp.exp(u3 - m)
        den = jnp.sum(e, axis=1, keepdims=True)
        sm = e / den
        sc = jnp.max(sm, axis=0)
        ts_ref[0, pl.ds(ck * _TJ, _TJ)] = jnp.sum(sc, axis=0)
    ts = ts_ref[0, :]

    tsr = ts[:, None]
    tsc = ts[None, :]
    row_i = lax.broadcasted_iota(jnp.int32, (_T, _T), 0)
    col_i = lax.broadcasted_iota(jnp.int32, (_T, _T), 1)
    gt = (tsc > tsr) | ((tsc == tsr) & (col_i < row_i))
    rank = jnp.sum(gt.astype(jnp.float32), axis=1).astype(jnp.int32)

    kio = lax.broadcasted_iota(jnp.int32, (_N, _T), 0)
    jio = lax.broadcasted_iota(jnp.int32, (_N, _T), 1)
    sel = rank[None, :] == kio
    perm = jnp.sum(jnp.where(sel, jio, 0), axis=1)

    b = pl.program_id(0)
    ck = perm[:, None] // _TJ
    jloc = perm[:, None] % _TJ
    lio = lax.broadcasted_iota(jnp.int32, (_N, _LT), 1)
    o_ref[0] = b * (_T * _LT) + ck * (_LT * _TJ) + lio * _TJ + jloc


def _finalize_kernel(t_ref, c_ref, o_ref):
    t = t_ref[...]
    c = c_ref[...]
    u = lax.dot_general(t, c, (((1,), (1,)), ((), ())),
                        preferred_element_type=jnp.float32)
    u3 = u.reshape(_N, _LT, _LC)
    nrm = jnp.sqrt(jnp.sum(u3 * u3, axis=1, keepdims=True))
    o_ref[0] = u3 / nrm


def kernel(claim, targets, embeddings):
    B, LC = claim.shape
    _, T, LT = targets.shape
    D = embeddings.shape[1]
    N = _N

    t_g, c_g = _sc_gather_all(embeddings, targets.astype(jnp.int32),
                              claim.astype(jnp.int32))

    idxf = pl.pallas_call(
        _score_topk_kernel,
        grid=(B,),
        in_specs=[
            pl.BlockSpec((T * LT, D), lambda b: (b, 0)),
            pl.BlockSpec((LC, D), lambda b: (b, 0)),
        ],
        out_specs=pl.BlockSpec((1, N, LT), lambda b: (b, 0, 0)),
        out_shape=jax.ShapeDtypeStruct((B, N, LT), jnp.int32),
        scratch_shapes=[pltpu.VMEM((1, _T), jnp.float32)],
    )(t_g, c_g)

    sel_ids = idxf.reshape(-1)
    t_sel = _sc_gather(t_g, sel_ids, chunk=B * N * LT // _NW)

    fin = pl.pallas_call(
        _finalize_kernel,
        grid=(B,),
        in_specs=[
            pl.BlockSpec((N * LT, D), lambda b: (b, 0)),
            pl.BlockSpec((LC, D), lambda b: (b, 0)),
        ],
        out_specs=pl.BlockSpec((1, N, LT, LC), lambda b: (b, 0, 0, 0)),
        out_shape=jax.ShapeDtypeStruct((B, N, LT, LC), jnp.float32),
    )(t_sel, c_g)

    return fin.transpose(0, 1, 3, 2)


# speedup vs baseline: 3.2788x; 1.0095x over previous
import functools

import jax
import jax.numpy as jnp
from jax import lax
from jax.experimental import pallas as pl
from jax.experimental.pallas import tpu as pltpu
from jax.experimental.pallas import tpu_sc as plsc

_NC = 2
_NS = 16
_NW = _NC * _NS

_T = 512
_LT = 32
_LC = 16
_N = 64
_TJ = 128
_NCK = _T // _TJ
_CHJ = 64


def _sc_gather_all(emb, targets, claim):
    B, T, LT = targets.shape
    _, LC = claim.shape
    D = emb.shape[1]
    M = B * T * LT
    jw = T // 2
    n_steps = jw // _CHJ
    cw = B * LC // _NW
    mesh = plsc.VectorSubcoreMesh(
        core_axis_name="c", subcore_axis_name="s", num_cores=_NC,
        num_subcores=_NS)

    @functools.partial(
        pl.kernel,
        mesh=mesh,
        compiler_params=pltpu.CompilerParams(use_tc_tiling_on_sc=False),
        out_type=(jax.ShapeDtypeStruct((M, D), jnp.float32),
                  jax.ShapeDtypeStruct((B * LC, D), jnp.float32)),
        scratch_types=[
            pltpu.VMEM((_CHJ, LT), jnp.int32),
            pltpu.VMEM((_CHJ * LT,), jnp.int32),
            pltpu.VMEM((_CHJ * LT,), jnp.int32),
            pltpu.VMEM((_CHJ * LT, D), jnp.float32),
            pltpu.VMEM((cw,), jnp.int32),
            pltpu.VMEM((cw, D), jnp.float32),
            pltpu.SemaphoreType.DMA,
        ],
    )
    def gather_kernel(tgt_hbm, clm_hbm, src_hbm, out_hbm, outc_hbm,
                      idx2_v, idx1_v, dst_v, rows_v, cidx_v, crows_v, sem):
        wid = lax.axis_index("s") * _NC + lax.axis_index("c")
        b0 = wid // 2
        half = wid % 2

        cbase = wid * cw
        pltpu.sync_copy(clm_hbm.at[b0, pl.ds(half * cw, cw)], cidx_v)
        pltpu.async_copy(src_hbm.at[cidx_v], crows_v, sem).wait()
        pltpu.sync_copy(crows_v, outc_hbm.at[pl.ds(cbase, cw)])

        j0 = half * jw
        lane = lax.iota(jnp.int32, 16)

        def body(i, carry):
            pltpu.sync_copy(tgt_hbm.at[b0, pl.ds(j0 + i * _CHJ, _CHJ), :],
                            idx2_v)

            def unpack(r, c2):
                j = j0 + i * _CHJ + r
                s = (b0 * (T * LT) + (j // _TJ) * (_LT * _TJ) + (j % _TJ))
                for c in range(LT // 16):
                    co = c * 16
                    idx1_v[pl.ds(r * LT + co, 16)] = idx2_v[r, pl.ds(co, 16)]
                    dst_v[pl.ds(r * LT + co, 16)] = s + (lane + co) * _TJ
                return c2

            lax.fori_loop(0, _CHJ, unpack, 0)
            pltpu.async_copy(src_hbm.at[idx1_v], rows_v, sem).wait()
            pltpu.async_copy(rows_v, out_hbm.at[dst_v], sem).wait()
            return carry

        lax.fori_loop(0, n_steps, body, 0)

    return gather_kernel(targets, claim, emb)


def _sc_gather(src, ids, chunk):
    M = ids.shape[0]
    D = src.shape[1]
    per_w = M // _NW
    n_chunks = per_w // chunk
    mesh = plsc.VectorSubcoreMesh(
        core_axis_name="c", subcore_axis_name="s", num_cores=_NC,
        num_subcores=_NS)

    @functools.partial(
        pl.kernel,
        mesh=mesh,
        compiler_params=pltpu.CompilerParams(use_tc_tiling_on_sc=False),
        out_type=jax.ShapeDtypeStruct((M, D), jnp.float32),
        scratch_types=[
            pltpu.VMEM((chunk,), jnp.int32),
            pltpu.VMEM((chunk, D), jnp.float32),
            pltpu.SemaphoreType.DMA,
        ],
    )
    def gather_kernel(ids_hbm, src_hbm, out_hbm, idx_v, rows_v, sem):
        wid = lax.axis_index("s") * _NC + lax.axis_index("c")
        base = wid * per_w

        def body(i, carry):
            off = base + i * chunk
            pltpu.sync_copy(ids_hbm.at[pl.ds(off, chunk)], idx_v)
            pltpu.async_copy(src_hbm.at[idx_v], rows_v, sem).wait()
            pltpu.sync_copy(rows_v, out_hbm.at[pl.ds(off, chunk)])
            return carry

        lax.fori_loop(0, n_chunks, body, 0)

    return gather_kernel(ids, src)


def _score_topk_kernel(t_hbm, c_hbm, o_ref, tbuf, cbuf, ts_ref, sem0, sem1):
    b = pl.program_id(0)
    base = b * (_T * _LT)
    CH = _LT * _TJ
    sems = (sem0, sem1)
    pltpu.make_async_copy(c_hbm.at[pl.ds(b * _LC, _LC), :], cbuf, sem1).start()
    pltpu.make_async_copy(t_hbm.at[pl.ds(base, CH), :], tbuf.at[0],
                          sem0).start()
    pltpu.make_async_copy(c_hbm.at[pl.ds(b * _LC, _LC), :], cbuf, sem1).wait()
    c = cbuf[...]
    for ck in range(_NCK):
        if ck + 1 < _NCK:
            pltpu.make_async_copy(
                t_hbm.at[pl.ds(base + (ck + 1) * CH, CH), :],
                tbuf.at[(ck + 1) % 2], sems[(ck + 1) % 2]).start()
        pltpu.make_async_copy(t_hbm.at[pl.ds(base + ck * CH, CH), :],
                              tbuf.at[ck % 2], sems[ck % 2]).wait()
        t = tbuf[ck % 2]
        u = lax.dot_general(c, t, (((1,), (1,)), ((), ())),
                            preferred_element_type=jnp.float32)
        u3 = u.reshape(_LC, _LT, _TJ)
        m = jnp.max(u3, axis=1, keepdims=True)
        e = jnp.exp(u3 - m)
        den = jnp.sum(e, axis=1, keepdims=True)
        sm = e / den
        sc = jnp.max(sm, axis=0)
        ts_ref[0, pl.ds(ck * _TJ, _TJ)] = jnp.sum(sc, axis=0)

    tsc = ts_ref[0:1, :]
    tsr = jnp.transpose(tsc)
    col_i = lax.broadcasted_iota(jnp.int32, (1, _T), 1)
    row_i = lax.broadcasted_iota(jnp.int32, (_T, 1), 0)
    gt = (tsc > tsr) | ((tsc == tsr) & (col_i < row_i))
    rank = jnp.sum(gt.astype(jnp.float32), axis=1, keepdims=True)

    kio = lax.broadcasted_iota(jnp.int32, (1, _N), 1).astype(jnp.float32)
    sel = (rank == kio).astype(jnp.float32)
    jval = row_i.astype(jnp.float32)
    perm = lax.dot_general(sel, jval, (((0,), (0,)), ((), ())),
                           precision=jax.lax.Precision.HIGHEST,
                           preferred_element_type=jnp.float32)
    permi = (perm + 0.5).astype(jnp.int32)
    lio = lax.broadcasted_iota(jnp.int32, (1, _LT), 1)
    o_ref[0] = (base + (permi // _TJ) * (_LT * _TJ) + (permi % _TJ)
                + lio * _TJ)


def _finalize_kernel(t_hbm, c_hbm, o_ref, tbuf, cbuf, sem0, sem1):
    b = pl.program_id(0)
    pltpu.make_async_copy(c_hbm.at[pl.ds(b * _LC, _LC), :], cbuf, sem1).start()
    pltpu.make_async_copy(t_hbm.at[pl.ds(b * _N * _LT, _N * _LT), :], tbuf,
                          sem0).start()
    pltpu.make_async_copy(c_hbm.at[pl.ds(b * _LC, _LC), :], cbuf, sem1).wait()
    pltpu.make_async_copy(t_hbm.at[pl.ds(b * _N * _LT, _N * _LT), :], tbuf,
                          sem0).wait()
    t = tbuf[...]
    c = cbuf[...]
    u = lax.dot_general(t, c, (((1,), (1,)), ((), ())),
                        preferred_element_type=jnp.float32)
    u3 = u.reshape(_N, _LT, _LC)
    nrm = jnp.sqrt(jnp.sum(u3 * u3, axis=1, keepdims=True))
    o_ref[0] = u3 / nrm


def kernel(claim, targets, embeddings):
    B, LC = claim.shape
    _, T, LT = targets.shape
    D = embeddings.shape[1]
    N = _N

    t_g, c_g = _sc_gather_all(embeddings, targets.astype(jnp.int32),
                              claim.astype(jnp.int32))

    idxf = pl.pallas_call(
        _score_topk_kernel,
        grid=(B,),
        in_specs=[
            pl.BlockSpec(memory_space=pltpu.MemorySpace.HBM),
            pl.BlockSpec(memory_space=pltpu.MemorySpace.HBM),
        ],
        out_specs=pl.BlockSpec((1, N, LT), lambda b: (b, 0, 0)),
        out_shape=jax.ShapeDtypeStruct((B, N, LT), jnp.int32),
        scratch_shapes=[
            pltpu.VMEM((2, _LT * _TJ, D), jnp.float32),
            pltpu.VMEM((LC, D), jnp.float32),
            pltpu.VMEM((1, _T), jnp.float32),
            pltpu.SemaphoreType.DMA,
            pltpu.SemaphoreType.DMA,
        ],
    )(t_g, c_g)

    sel_ids = idxf.reshape(-1)
    t_sel = _sc_gather(t_g, sel_ids, chunk=B * N * LT // _NW)

    fin = pl.pallas_call(
        _finalize_kernel,
        grid=(B,),
        in_specs=[
            pl.BlockSpec(memory_space=pltpu.MemorySpace.HBM),
            pl.BlockSpec(memory_space=pltpu.MemorySpace.HBM),
        ],
        out_specs=pl.BlockSpec((1, N, LT, LC), lambda b: (b, 0, 0, 0)),
        out_shape=jax.ShapeDtypeStruct((B, N, LT, LC), jnp.float32),
        scratch_shapes=[
            pltpu.VMEM((N * LT, D), jnp.float32),
            pltpu.VMEM((LC, D), jnp.float32),
            pltpu.SemaphoreType.DMA,
            pltpu.SemaphoreType.DMA,
        ],
    )(t_sel, c_g)

    return fin.transpose(0, 1, 3, 2)


# speedup vs baseline: 4.2953x; 1.3100x over previous
import functools

import jax
import jax.numpy as jnp
from jax import lax
from jax.experimental import pallas as pl
from jax.experimental.pallas import tpu as pltpu
from jax.experimental.pallas import tpu_sc as plsc

_NC = 2
_NS = 16
_NW = _NC * _NS

_T = 512
_LT = 32
_LC = 16
_N = 64
_TJ = 128
_NCK = _T // _TJ
_CHJ = 64


def _sc_gather_all(emb, targets, claim):
    B, T, LT = targets.shape
    _, LC = claim.shape
    D = emb.shape[1]
    M = B * T * LT
    jw = T // 2
    n_steps = jw // _CHJ
    cw = B * LC // _NW
    mesh = plsc.VectorSubcoreMesh(
        core_axis_name="c", subcore_axis_name="s", num_cores=_NC,
        num_subcores=_NS)

    @functools.partial(
        pl.kernel,
        mesh=mesh,
        compiler_params=pltpu.CompilerParams(use_tc_tiling_on_sc=False),
        out_type=(jax.ShapeDtypeStruct((M, D), jnp.float32),
                  jax.ShapeDtypeStruct((B * LC, D), jnp.float32)),
        scratch_types=[
            pltpu.VMEM((_CHJ, LT), jnp.int32),
            pltpu.VMEM((_CHJ * LT,), jnp.int32),
            pltpu.VMEM((_CHJ * LT,), jnp.int32),
            pltpu.VMEM((_CHJ * LT, D), jnp.float32),
            pltpu.VMEM((cw,), jnp.int32),
            pltpu.VMEM((cw, D), jnp.float32),
            pltpu.SemaphoreType.DMA,
        ],
    )
    def gather_kernel(tgt_hbm, clm_hbm, src_hbm, out_hbm, outc_hbm,
                      idx2_v, idx1_v, dst_v, rows_v, cidx_v, crows_v, sem):
        wid = lax.axis_index("s") * _NC + lax.axis_index("c")
        b0 = wid // 2
        half = wid % 2

        cbase = wid * cw
        pltpu.sync_copy(clm_hbm.at[b0, pl.ds(half * cw, cw)], cidx_v)
        pltpu.async_copy(src_hbm.at[cidx_v], crows_v, sem).wait()
        pltpu.sync_copy(crows_v, outc_hbm.at[pl.ds(cbase, cw)])

        j0 = half * jw
        lane = lax.iota(jnp.int32, 16)

        def body(i, carry):
            pltpu.sync_copy(tgt_hbm.at[b0, pl.ds(j0 + i * _CHJ, _CHJ), :],
                            idx2_v)

            def unpack(r, c2):
                j = j0 + i * _CHJ + r
                s = (b0 * (T * LT) + (j // _TJ) * (_LT * _TJ) + (j % _TJ))
                for c in range(LT // 16):
                    co = c * 16
                    idx1_v[pl.ds(r * LT + co, 16)] = idx2_v[r, pl.ds(co, 16)]
                    dst_v[pl.ds(r * LT + co, 16)] = s + (lane + co) * _TJ
                return c2

            lax.fori_loop(0, _CHJ, unpack, 0)
            pltpu.async_copy(src_hbm.at[idx1_v], rows_v, sem).wait()
            pltpu.async_copy(rows_v, out_hbm.at[dst_v], sem).wait()
            return carry

        lax.fori_loop(0, n_steps, body, 0)

    return gather_kernel(targets, claim, emb)


def _sc_gather(src, ids, chunk):
    M = ids.shape[0]
    D = src.shape[1]
    per_w = M // _NW
    n_chunks = per_w // chunk
    mesh = plsc.VectorSubcoreMesh(
        core_axis_name="c", subcore_axis_name="s", num_cores=_NC,
        num_subcores=_NS)

    @functools.partial(
        pl.kernel,
        mesh=mesh,
        compiler_params=pltpu.CompilerParams(use_tc_tiling_on_sc=False),
        out_type=jax.ShapeDtypeStruct((M, D), jnp.float32),
        scratch_types=[
            pltpu.VMEM((chunk,), jnp.int32),
            pltpu.VMEM((chunk, D), jnp.float32),
            pltpu.SemaphoreType.DMA,
        ],
    )
    def gather_kernel(ids_hbm, src_hbm, out_hbm, idx_v, rows_v, sem):
        wid = lax.axis_index("s") * _NC + lax.axis_index("c")
        base = wid * per_w

        def body(i, carry):
            off = base + i * chunk
            pltpu.sync_copy(ids_hbm.at[pl.ds(off, chunk)], idx_v)
            pltpu.async_copy(src_hbm.at[idx_v], rows_v, sem).wait()
            pltpu.sync_copy(rows_v, out_hbm.at[pl.ds(off, chunk)])
            return carry

        lax.fori_loop(0, n_chunks, body, 0)

    return gather_kernel(ids, src)


def _score_topk_kernel(t_hbm, c_hbm, o_ref, tbuf, cbuf, w_ref, ts_ref,
                       sem0, sem1):
    b = pl.program_id(0)
    base = b * (_T * _LT)
    CH = _LT * _TJ
    sems = (sem0, sem1)
    pltpu.make_async_copy(c_hbm.at[pl.ds(b * _LC, _LC), :], cbuf, sem1).start()
    pltpu.make_async_copy(t_hbm.at[pl.ds(base // 4, CH // 4), :], tbuf.at[0],
                          sem0).start()
    pltpu.make_async_copy(c_hbm.at[pl.ds(b * _LC, _LC), :], cbuf, sem1).wait()
    c = cbuf[...]
    for ck in range(_NCK):
        if ck + 1 < _NCK:
            pltpu.make_async_copy(
                t_hbm.at[pl.ds((base + (ck + 1) * CH) // 4, CH // 4), :],
                tbuf.at[(ck + 1) % 2], sems[(ck + 1) % 2]).start()
        pltpu.make_async_copy(t_hbm.at[pl.ds((base + ck * CH) // 4, CH // 4),
                                       :],
                              tbuf.at[ck % 2], sems[ck % 2]).wait()
        t = tbuf[ck % 2]
        for q in range(4):
            uq = lax.dot_general(c, t[:, q * 32:(q + 1) * 32],
                                 (((1,), (1,)), ((), ())),
                                 preferred_element_type=jnp.float32)
            w_ref[:, :, pl.ds(q * 32, 32)] = uq.reshape(_LC, _LT, 32)
        u3 = w_ref[...]
        m = jnp.max(u3, axis=1, keepdims=True)
        e = jnp.exp(u3 - m)
        den = jnp.sum(e, axis=1, keepdims=True)
        sm = e / den
        sc = jnp.max(sm, axis=0)
        ts_ref[0, pl.ds(ck * _TJ, _TJ)] = jnp.sum(sc, axis=0)

    tsc = ts_ref[0:1, :]
    tsr = jnp.transpose(tsc)
    col_i = lax.broadcasted_iota(jnp.int32, (1, _T), 1)
    row_i = lax.broadcasted_iota(jnp.int32, (_T, 1), 0)
    col_j = (col_i // _TJ) * _TJ + 4 * (col_i % 32) + (col_i % _TJ) // 32
    row_j = (row_i // _TJ) * _TJ + 4 * (row_i % 32) + (row_i % _TJ) // 32
    gt = (tsc > tsr) | ((tsc == tsr) & (col_j < row_j))
    rank = jnp.sum(gt.astype(jnp.float32), axis=1, keepdims=True)

    kio = lax.broadcasted_iota(jnp.int32, (1, _N), 1).astype(jnp.float32)
    sel = (rank == kio).astype(jnp.float32)
    jval = row_j.astype(jnp.float32)
    perm = lax.dot_general(sel, jval, (((0,), (0,)), ((), ())),
                           precision=jax.lax.Precision.HIGHEST,
                           preferred_element_type=jnp.float32)
    permi = (perm + 0.5).astype(jnp.int32)
    lio = lax.broadcasted_iota(jnp.int32, (1, _LT), 1)
    o_ref[0] = (base + (permi // _TJ) * (_LT * _TJ) + (permi % _TJ)
                + lio * _TJ)


def _finalize_kernel(t_hbm, c_hbm, o_ref, tbuf, cbuf, sem0, sem1):
    b = pl.program_id(0)
    pltpu.make_async_copy(c_hbm.at[pl.ds(b * _LC, _LC), :], cbuf, sem1).start()
    pltpu.make_async_copy(t_hbm.at[pl.ds(b * _N * _LT, _N * _LT), :], tbuf,
                          sem0).start()
    pltpu.make_async_copy(c_hbm.at[pl.ds(b * _LC, _LC), :], cbuf, sem1).wait()
    pltpu.make_async_copy(t_hbm.at[pl.ds(b * _N * _LT, _N * _LT), :], tbuf,
                          sem0).wait()
    t = tbuf[...]
    c = cbuf[...]
    u = lax.dot_general(t, c, (((1,), (1,)), ((), ())),
                        preferred_element_type=jnp.float32)
    u3 = u.reshape(_N, _LT, _LC)
    nrm = jnp.sqrt(jnp.sum(u3 * u3, axis=1, keepdims=True))
    o_ref[0] = u3 / nrm


def kernel(claim, targets, embeddings):
    B, LC = claim.shape
    _, T, LT = targets.shape
    D = embeddings.shape[1]
    N = _N

    t_g, c_g = _sc_gather_all(embeddings, targets.astype(jnp.int32),
                              claim.astype(jnp.int32))

    idxf = pl.pallas_call(
        _score_topk_kernel,
        grid=(B,),
        in_specs=[
            pl.BlockSpec(memory_space=pltpu.MemorySpace.HBM),
            pl.BlockSpec(memory_space=pltpu.MemorySpace.HBM),
        ],
        out_specs=pl.BlockSpec((1, N, LT), lambda b: (b, 0, 0)),
        out_shape=jax.ShapeDtypeStruct((B, N, LT), jnp.int32),
        scratch_shapes=[
            pltpu.VMEM((2, _LT * _TJ // 4, 4 * D), jnp.float32),
            pltpu.VMEM((LC, D), jnp.float32),
            pltpu.VMEM((_LC, _LT, _TJ), jnp.float32),
            pltpu.VMEM((1, _T), jnp.float32),
            pltpu.SemaphoreType.DMA,
            pltpu.SemaphoreType.DMA,
        ],
    )(t_g.reshape(B * T * LT // 4, 4 * D), c_g)

    sel_ids = idxf.reshape(-1)
    t_sel = _sc_gather(t_g, sel_ids, chunk=B * N * LT // _NW)

    fin = pl.pallas_call(
        _finalize_kernel,
        grid=(B,),
        in_specs=[
            pl.BlockSpec(memory_space=pltpu.MemorySpace.HBM),
            pl.BlockSpec(memory_space=pltpu.MemorySpace.HBM),
        ],
        out_specs=pl.BlockSpec((1, N, LT, LC), lambda b: (b, 0, 0, 0)),
        out_shape=jax.ShapeDtypeStruct((B, N, LT, LC), jnp.float32),
        scratch_shapes=[
            pltpu.VMEM((N * LT, D), jnp.float32),
            pltpu.VMEM((LC, D), jnp.float32),
            pltpu.SemaphoreType.DMA,
            pltpu.SemaphoreType.DMA,
        ],
    )(t_sel, c_g)

    return fin.transpose(0, 1, 3, 2)
